# baseline (speedup 1.0000x reference)
import functools

import jax
import jax.numpy as jnp
from jax import lax
from jax.experimental import pallas as pl
from jax.experimental.pallas import tpu as pltpu
from jax.experimental.pallas import tpu_sc as plsc

N = 100000
E = 3200000
G = 64
D = 16

LANES = 128
NBLK = E // LANES
KG = 8
NTILES = 16
STRIPE = 6256
STRIPE_LAST = N - STRIPE * (NTILES - 1)
GE = 512
NSG = E // GE
SG_LO = NSG // NTILES
N_HI_TILES2 = NSG - SG_LO * NTILES
NPAIRS = (SG_LO + 1) // 2

RBLK = 2000
NGRID = N // RBLK


def _lrelu(x):
    return jnp.where(x >= 0, x, 0.01 * x)


def _edge_body(eiL, hmL, eiR, hmR, zrows, aggL, aggR,
               sbuf0, sbuf1, dbuf0, dbuf1, rows0, rows1, aggsh,
               sem_is0, sem_is1, sem_id0, sem_id1,
               sem_g0, sem_g1, sem_s0, sem_s1):
    c = lax.axis_index("c")
    s = lax.axis_index("s")

    @pl.when(s < NTILES - 1)
    def _():
        pltpu.sync_copy(zrows, aggsh.at[pl.ds(s * STRIPE, STRIPE)])

    @pl.when(s == NTILES - 1)
    def _():
        pltpu.sync_copy(zrows.at[pl.ds(0, STRIPE_LAST)],
                        aggsh.at[pl.ds(s * STRIPE, STRIPE_LAST)])

    plsc.subcore_barrier()

    sbufs = (sbuf0, sbuf1)
    dbufs = (dbuf0, dbuf1)
    rowss = (rows0, rows1)
    sems_is = (sem_is0, sem_is1)
    sems_id = (sem_id0, sem_id1)
    sems_g = (sem_g0, sem_g1)
    sems_s = (sem_s0, sem_s1)

    def side(ei, hm):
        def fire_isrc(p, e0):
            pltpu.async_copy(ei.at[0, pl.ds(e0, GE)], sbufs[p], sems_is[p])

        def fire_idst(p, e0):
            pltpu.async_copy(ei.at[1, pl.ds(e0, GE)], dbufs[p], sems_id[p])

        def wait_is(p):
            pltpu.make_async_copy(ei.at[0, pl.ds(0, GE)], sbufs[p],
                                  sems_is[p]).wait()

        def wait_id(p):
            pltpu.make_async_copy(ei.at[1, pl.ds(0, GE)], dbufs[p],
                                  sems_id[p]).wait()

        def fire_g(p):
            pltpu.async_copy(hm.at[sbufs[p]], rowss[p], sems_g[p])

        def wait_g(p):
            pltpu.make_async_copy(hm.at[sbufs[p]], rowss[p],
                                  sems_g[p]).wait()

        def fire_s(p):
            pltpu.async_copy(rowss[p], aggsh.at[dbufs[p]], sems_s[p],
                             add=True)

        def wait_s(p):
            pltpu.make_async_copy(rowss[p], aggsh.at[dbufs[p]],
                                  sems_s[p]).wait()

        base_e = (s * SG_LO + jnp.minimum(s, N_HI_TILES2)) * GE
        has_extra = s < N_HI_TILES2
        ng2 = jnp.where(s < N_HI_TILES2, SG_LO + 1, SG_LO)

        fire_isrc(0, base_e)
        fire_idst(0, base_e)
        fire_isrc(1, base_e + GE)
        fire_idst(1, base_e + GE)
        wait_is(0)
        fire_g(0)

        def body(k, carry):
            a_e = base_e + 2 * k * GE
            next0 = 2 * k + 2 < ng2
            next1 = 2 * k + 3 < ng2
            wait_g(0)

            @pl.when(next0)
            def _():
                fire_isrc(0, a_e + 2 * GE)

            wait_is(1)
            wait_id(1)
            fire_g(1)
            wait_id(0)
            fire_s(0)
            wait_g(1)
            fire_s(1)
            wait_s(0)

            @pl.when(next0)
            def _():
                fire_idst(0, a_e + 2 * GE)
                wait_is(0)
                fire_g(0)

            wait_s(1)

            @pl.when(next1)
            def _():
                fire_isrc(1, a_e + 3 * GE)
                fire_idst(1, a_e + 3 * GE)

            return carry

        lax.fori_loop(0, NPAIRS, body, 0)

        @pl.when(has_extra)
        def _():
            wait_id(0)
            wait_g(0)
            fire_s(0)
            wait_s(0)

    @pl.when(c == 0)
    def _():
        side(eiL, hmL)

    @pl.when(c == 1)
    def _():
        side(eiR, hmR)

    plsc.subcore_barrier()

    def copy_out(out):
        r0 = s * STRIPE

        @pl.when(s < NTILES - 1)
        def _():
            pltpu.sync_copy(aggsh.at[pl.ds(r0, STRIPE)],
                            out.at[pl.ds(r0, STRIPE)])

        @pl.when(s == NTILES - 1)
        def _():
            pltpu.sync_copy(aggsh.at[pl.ds(r0, STRIPE_LAST)],
                            out.at[pl.ds(r0, STRIPE_LAST)])

    @pl.when(c == 0)
    def _():
        copy_out(aggL)

    @pl.when(c == 1)
    def _():
        copy_out(aggR)


@functools.cache
def _edge_pass_fn():
    return pl.kernel(
        _edge_body,
        out_type=(jax.ShapeDtypeStruct((N, D), jnp.float32),
                  jax.ShapeDtypeStruct((N, D), jnp.float32)),
        mesh=plsc.VectorSubcoreMesh(core_axis_name="c", subcore_axis_name="s",
                                    num_cores=2, num_subcores=NTILES),
        scratch_types=[
            pltpu.VMEM((GE,), jnp.int32),
            pltpu.VMEM((GE,), jnp.int32),
            pltpu.VMEM((GE,), jnp.int32),
            pltpu.VMEM((GE,), jnp.int32),
            pltpu.VMEM((GE, D), jnp.float32),
            pltpu.VMEM((GE, D), jnp.float32),
            pltpu.VMEM_SHARED((N, D), jnp.float32),
        ] + [pltpu.SemaphoreType.DMA] * 8,
        compiler_params=pltpu.CompilerParams(use_tc_tiling_on_sc=False),
    )


def _edge_pass(*args):
    return _edge_pass_fn()(*args)


def _hm1_body(sp_ref, emb_ref, wm_ref, bm_ref, out_ref):
    table = _lrelu(jnp.dot(emb_ref[...], wm_ref[...],
                           preferred_element_type=jnp.float32) + bm_ref[...])
    sp = sp_ref[...]
    h = jnp.zeros((RBLK, D), jnp.float32)
    for sidx in range(5):
        h = jnp.where(sp == sidx, table[sidx:sidx + 1, :], h)
    out_ref[...] = h


def _hm1_call(sp, emb_p, wm, bm):
    return pl.pallas_call(
        _hm1_body,
        grid=(NGRID,),
        in_specs=[
            pl.BlockSpec((RBLK, 1), lambda i: (i, 0)),
            pl.BlockSpec((8, D), lambda i: (0, 0)),
            pl.BlockSpec((D, D), lambda i: (0, 0)),
            pl.BlockSpec((1, D), lambda i: (0, 0)),
        ],
        out_specs=pl.BlockSpec((RBLK, D), lambda i: (i, 0)),
        out_shape=jax.ShapeDtypeStruct((N, D), jnp.float32),
    )(sp, emb_p, wm, bm)


def _mid_body(sp_ref, emb_ref, agg_ref, wu_ref, bu_ref, wm_ref, bm_ref,
              h1_ref, hm2_ref):
    sp = sp_ref[...]
    emb = emb_ref[...]
    h0 = jnp.zeros((RBLK, D), jnp.float32)
    for sidx in range(5):
        h0 = jnp.where(sp == sidx, emb[sidx:sidx + 1, :], h0)
    upd = _lrelu(jnp.dot(agg_ref[...], wu_ref[...],
                         preferred_element_type=jnp.float32) + bu_ref[...])
    h1 = h0 + upd
    h1_ref[...] = h1
    hm2_ref[...] = _lrelu(jnp.dot(h1, wm_ref[...],
                                  preferred_element_type=jnp.float32)
                          + bm_ref[...])


def _mid_call(sp, emb_p, agg1, wu, bu, wm, bm):
    return pl.pallas_call(
        _mid_body,
        grid=(NGRID,),
        in_specs=[
            pl.BlockSpec((RBLK, 1), lambda i: (i, 0)),
            pl.BlockSpec((8, D), lambda i: (0, 0)),
            pl.BlockSpec((RBLK, D), lambda i: (i, 0)),
            pl.BlockSpec((D, D), lambda i: (0, 0)),
            pl.BlockSpec((1, D), lambda i: (0, 0)),
            pl.BlockSpec((D, D), lambda i: (0, 0)),
            pl.BlockSpec((1, D), lambda i: (0, 0)),
        ],
        out_specs=[
            pl.BlockSpec((RBLK, D), lambda i: (i, 0)),
            pl.BlockSpec((RBLK, D), lambda i: (i, 0)),
        ],
        out_shape=[jax.ShapeDtypeStruct((N, D), jnp.float32),
                   jax.ShapeDtypeStruct((N, D), jnp.float32)],
    )(sp, emb_p, agg1, wu, bu, wm, bm)


def _pool_body(h1_ref, agg_ref, wu_ref, bu_ref, batch_ref, out_ref):
    i = pl.program_id(0)
    upd = _lrelu(jnp.dot(agg_ref[...], wu_ref[...],
                         preferred_element_type=jnp.float32) + bu_ref[...])
    h2 = h1_ref[...] + upd
    gids = lax.broadcasted_iota(jnp.int32, (1, G), 1)
    mask = (batch_ref[...] == gids).astype(jnp.float32)
    pool = lax.dot_general(mask, h2, (((0,), (0,)), ((), ())),
                           precision=lax.Precision.HIGHEST,
                           preferred_element_type=jnp.float32)

    @pl.when(i == 0)
    def _():
        out_ref[...] = jnp.zeros((G, D), jnp.float32)

    out_ref[...] += pool


def _pool_call(h1, agg2, wu, bu, batch):
    return pl.pallas_call(
        _pool_body,
        grid=(NGRID,),
        in_specs=[
            pl.BlockSpec((RBLK, D), lambda i: (i, 0)),
            pl.BlockSpec((RBLK, D), lambda i: (i, 0)),
            pl.BlockSpec((D, D), lambda i: (0, 0)),
            pl.BlockSpec((1, D), lambda i: (0, 0)),
            pl.BlockSpec((RBLK, 1), lambda i: (i, 0)),
        ],
        out_specs=pl.BlockSpec((G, D), lambda i: (0, 0)),
        out_shape=jax.ShapeDtypeStruct((G, D), jnp.float32),
    )(h1, agg2, wu, bu, batch)


def _head_body(xl_ref, xr_ref, f_ref, wa_ref, wb_ref, wc_ref, b1_ref,
               w2_ref, b2_ref, out_ref):
    x = (jnp.dot(xl_ref[...], wa_ref[...], preferred_element_type=jnp.float32)
         + jnp.dot(xr_ref[...], wb_ref[...], preferred_element_type=jnp.float32)
         + f_ref[...] * wc_ref[...]
         + b1_ref[...])
    x = _lrelu(x)
    out_ref[...] = jnp.dot(x, w2_ref[...],
                           preferred_element_type=jnp.float32) + b2_ref[...]


def _head_call(xl, xr, force, wa, wb, wc, b1, w2, b2):
    return pl.pallas_call(
        _head_body,
        out_shape=jax.ShapeDtypeStruct((G, 1), jnp.float32),
    )(xl, xr, force, wa, wb, wc, b1, w2, b2)


def kernel(left_species, left_edge_index, left_batch,
           right_species, right_edge_index, right_batch, force,
           emb_L, Wm1_L, bm1_L, Wu1_L, bu1_L, Wm2_L, bm2_L, Wu2_L, bu2_L,
           emb_R, Wm1_R, bm1_R, Wu1_R, bu1_R, Wm2_R, bm2_R, Wu2_R, bu2_R,
           l1_W, l1_b, l2_W, l2_b):
    f32 = jnp.float32
    spL = left_species.astype(jnp.int32).reshape(N, 1)
    spR = right_species.astype(jnp.int32).reshape(N, 1)
    eiL = left_edge_index.astype(jnp.int32)
    eiR = right_edge_index.astype(jnp.int32)
    batL = left_batch.astype(jnp.int32).reshape(N, 1)
    batR = right_batch.astype(jnp.int32).reshape(N, 1)

    embL_p = jnp.zeros((8, D), f32).at[:5].set(emb_L.astype(f32))
    embR_p = jnp.zeros((8, D), f32).at[:5].set(emb_R.astype(f32))
    zrows = jnp.zeros((STRIPE, D), f32)

    def r1(b):
        return b.astype(f32).reshape(1, D)

    hm1L = _hm1_call(spL, embL_p, Wm1_L.astype(f32), r1(bm1_L))
    hm1R = _hm1_call(spR, embR_p, Wm1_R.astype(f32), r1(bm1_R))

    agg1L, agg1R = _edge_pass(eiL, hm1L, eiR, hm1R, zrows)

    h1L, hm2L = _mid_call(spL, embL_p, agg1L, Wu1_L.astype(f32), r1(bu1_L),
                          Wm2_L.astype(f32), r1(bm2_L))
    h1R, hm2R = _mid_call(spR, embR_p, agg1R, Wu1_R.astype(f32), r1(bu1_R),
                          Wm2_R.astype(f32), r1(bm2_R))

    agg2L, agg2R = _edge_pass(eiL, hm2L, eiR, hm2R, zrows)

    pooledL = _pool_call(h1L, agg2L, Wu2_L.astype(f32), r1(bu2_L), batL)
    pooledR = _pool_call(h1R, agg2R, Wu2_R.astype(f32), r1(bu2_R), batR)

    l1W = l1_W.astype(f32)
    out = _head_call(pooledL, pooledR, force.astype(f32),
                     l1W[0:D], l1W[D:2 * D], l1W[2 * D:2 * D + 1],
                     l1_b.astype(f32).reshape(1, D),
                     l2_W.astype(f32), l2_b.astype(f32).reshape(1, 1))
    return out


# speedup vs baseline: 1.0423x; 1.0423x over previous
import functools

import jax
import jax.numpy as jnp
from jax import lax
from jax.experimental import pallas as pl
from jax.experimental.pallas import tpu as pltpu
from jax.experimental.pallas import tpu_sc as plsc

N = 100000
E = 3200000
G = 64
D = 16

LANES = 128
NBLK = E // LANES
KG = 8
NTILES = 16
STRIPE = 6256
STRIPE_LAST = N - STRIPE * (NTILES - 1)
GE = 512
NSG = E // GE
SG_LO = NSG // NTILES
N_HI_TILES2 = NSG - SG_LO * NTILES
NPAIRS = (SG_LO + 1) // 2

RBLK = 10000
NGRID = N // RBLK


def _lrelu(x):
    return jnp.where(x >= 0, x, 0.01 * x)


def _edge_body(eiL, hmL, eiR, hmR, zrows, aggL, aggR,
               sbuf0, sbuf1, dbuf0, dbuf1, rows0, rows1, aggsh,
               sem_is0, sem_is1, sem_id0, sem_id1,
               sem_g0, sem_g1, sem_s0, sem_s1):
    c = lax.axis_index("c")
    s = lax.axis_index("s")

    @pl.when(s < NTILES - 1)
    def _():
        pltpu.sync_copy(zrows, aggsh.at[pl.ds(s * STRIPE, STRIPE)])

    @pl.when(s == NTILES - 1)
    def _():
        pltpu.sync_copy(zrows.at[pl.ds(0, STRIPE_LAST)],
                        aggsh.at[pl.ds(s * STRIPE, STRIPE_LAST)])

    plsc.subcore_barrier()

    sbufs = (sbuf0, sbuf1)
    dbufs = (dbuf0, dbuf1)
    rowss = (rows0, rows1)
    sems_is = (sem_is0, sem_is1)
    sems_id = (sem_id0, sem_id1)
    sems_g = (sem_g0, sem_g1)
    sems_s = (sem_s0, sem_s1)

    def side(ei, hm):
        def fire_isrc(p, e0):
            pltpu.async_copy(ei.at[0, pl.ds(e0, GE)], sbufs[p], sems_is[p])

        def fire_idst(p, e0):
            pltpu.async_copy(ei.at[1, pl.ds(e0, GE)], dbufs[p], sems_id[p])

        def wait_is(p):
            pltpu.make_async_copy(ei.at[0, pl.ds(0, GE)], sbufs[p],
                                  sems_is[p]).wait()

        def wait_id(p):
            pltpu.make_async_copy(ei.at[1, pl.ds(0, GE)], dbufs[p],
                                  sems_id[p]).wait()

        def fire_g(p):
            pltpu.async_copy(hm.at[sbufs[p]], rowss[p], sems_g[p])

        def wait_g(p):
            pltpu.make_async_copy(hm.at[sbufs[p]], rowss[p],
                                  sems_g[p]).wait()

        def fire_s(p):
            pltpu.async_copy(rowss[p], aggsh.at[dbufs[p]], sems_s[p],
                             add=True)

        def wait_s(p):
            pltpu.make_async_copy(rowss[p], aggsh.at[dbufs[p]],
                                  sems_s[p]).wait()

        base_e = (s * SG_LO + jnp.minimum(s, N_HI_TILES2)) * GE
        has_extra = s < N_HI_TILES2
        ng2 = jnp.where(s < N_HI_TILES2, SG_LO + 1, SG_LO)

        fire_isrc(0, base_e)
        fire_idst(0, base_e)
        fire_isrc(1, base_e + GE)
        fire_idst(1, base_e + GE)
        wait_is(0)
        fire_g(0)

        def body(k, carry):
            a_e = base_e + 2 * k * GE
            next0 = 2 * k + 2 < ng2
            next1 = 2 * k + 3 < ng2
            wait_g(0)

            @pl.when(next0)
            def _():
                fire_isrc(0, a_e + 2 * GE)

            wait_is(1)
            wait_id(1)
            fire_g(1)
            wait_id(0)
            fire_s(0)
            wait_g(1)
            fire_s(1)
            wait_s(0)

            @pl.when(next0)
            def _():
                fire_idst(0, a_e + 2 * GE)
                wait_is(0)
                fire_g(0)

            wait_s(1)

            @pl.when(next1)
            def _():
                fire_isrc(1, a_e + 3 * GE)
                fire_idst(1, a_e + 3 * GE)

            return carry

        lax.fori_loop(0, NPAIRS, body, 0)

        @pl.when(has_extra)
        def _():
            wait_id(0)
            wait_g(0)
            fire_s(0)
            wait_s(0)

    @pl.when(c == 0)
    def _():
        side(eiL, hmL)

    @pl.when(c == 1)
    def _():
        side(eiR, hmR)

    plsc.subcore_barrier()

    def copy_out(out):
        r0 = s * STRIPE

        @pl.when(s < NTILES - 1)
        def _():
            pltpu.sync_copy(aggsh.at[pl.ds(r0, STRIPE)],
                            out.at[pl.ds(r0, STRIPE)])

        @pl.when(s == NTILES - 1)
        def _():
            pltpu.sync_copy(aggsh.at[pl.ds(r0, STRIPE_LAST)],
                            out.at[pl.ds(r0, STRIPE_LAST)])

    @pl.when(c == 0)
    def _():
        copy_out(aggL)

    @pl.when(c == 1)
    def _():
        copy_out(aggR)


@functools.cache
def _edge_pass_fn():
    return pl.kernel(
        _edge_body,
        out_type=(jax.ShapeDtypeStruct((N, D), jnp.float32),
                  jax.ShapeDtypeStruct((N, D), jnp.float32)),
        mesh=plsc.VectorSubcoreMesh(core_axis_name="c", subcore_axis_name="s",
                                    num_cores=2, num_subcores=NTILES),
        scratch_types=[
            pltpu.VMEM((GE,), jnp.int32),
            pltpu.VMEM((GE,), jnp.int32),
            pltpu.VMEM((GE,), jnp.int32),
            pltpu.VMEM((GE,), jnp.int32),
            pltpu.VMEM((GE, D), jnp.float32),
            pltpu.VMEM((GE, D), jnp.float32),
            pltpu.VMEM_SHARED((N, D), jnp.float32),
        ] + [pltpu.SemaphoreType.DMA] * 8,
        compiler_params=pltpu.CompilerParams(use_tc_tiling_on_sc=False),
    )


def _edge_pass(*args):
    return _edge_pass_fn()(*args)


def _hm1_body(sp_ref, emb_ref, wm_ref, bm_ref, out_ref):
    table = _lrelu(jnp.dot(emb_ref[...], wm_ref[...],
                           preferred_element_type=jnp.float32) + bm_ref[...])
    sp = sp_ref[...]
    h = jnp.zeros((RBLK, D), jnp.float32)
    for sidx in range(5):
        h = jnp.where(sp == sidx, table[sidx:sidx + 1, :], h)
    out_ref[...] = h


def _hm1_call(sp, emb_p, wm, bm):
    return pl.pallas_call(
        _hm1_body,
        grid=(NGRID,),
        in_specs=[
            pl.BlockSpec((RBLK, 1), lambda i: (i, 0)),
            pl.BlockSpec((8, D), lambda i: (0, 0)),
            pl.BlockSpec((D, D), lambda i: (0, 0)),
            pl.BlockSpec((1, D), lambda i: (0, 0)),
        ],
        out_specs=pl.BlockSpec((RBLK, D), lambda i: (i, 0)),
        out_shape=jax.ShapeDtypeStruct((N, D), jnp.float32),
    )(sp, emb_p, wm, bm)


def _mid_body(sp_ref, emb_ref, agg_ref, wu_ref, bu_ref, wm_ref, bm_ref,
              h1_ref, hm2_ref):
    sp = sp_ref[...]
    emb = emb_ref[...]
    h0 = jnp.zeros((RBLK, D), jnp.float32)
    for sidx in range(5):
        h0 = jnp.where(sp == sidx, emb[sidx:sidx + 1, :], h0)
    upd = _lrelu(jnp.dot(agg_ref[...], wu_ref[...],
                         preferred_element_type=jnp.float32) + bu_ref[...])
    h1 = h0 + upd
    h1_ref[...] = h1
    hm2_ref[...] = _lrelu(jnp.dot(h1, wm_ref[...],
                                  preferred_element_type=jnp.float32)
                          + bm_ref[...])


def _mid_call(sp, emb_p, agg1, wu, bu, wm, bm):
    return pl.pallas_call(
        _mid_body,
        grid=(NGRID,),
        in_specs=[
            pl.BlockSpec((RBLK, 1), lambda i: (i, 0)),
            pl.BlockSpec((8, D), lambda i: (0, 0)),
            pl.BlockSpec((RBLK, D), lambda i: (i, 0)),
            pl.BlockSpec((D, D), lambda i: (0, 0)),
            pl.BlockSpec((1, D), lambda i: (0, 0)),
            pl.BlockSpec((D, D), lambda i: (0, 0)),
            pl.BlockSpec((1, D), lambda i: (0, 0)),
        ],
        out_specs=[
            pl.BlockSpec((RBLK, D), lambda i: (i, 0)),
            pl.BlockSpec((RBLK, D), lambda i: (i, 0)),
        ],
        out_shape=[jax.ShapeDtypeStruct((N, D), jnp.float32),
                   jax.ShapeDtypeStruct((N, D), jnp.float32)],
    )(sp, emb_p, agg1, wu, bu, wm, bm)


def _pool_body(h1_ref, agg_ref, wu_ref, bu_ref, batch_ref, out_ref):
    i = pl.program_id(0)
    upd = _lrelu(jnp.dot(agg_ref[...], wu_ref[...],
                         preferred_element_type=jnp.float32) + bu_ref[...])
    h2 = h1_ref[...] + upd
    gids = lax.broadcasted_iota(jnp.int32, (1, G), 1)
    mask = (batch_ref[...] == gids).astype(jnp.float32)
    pool = lax.dot_general(mask, h2, (((0,), (0,)), ((), ())),
                           precision=lax.Precision.HIGHEST,
                           preferred_element_type=jnp.float32)

    @pl.when(i == 0)
    def _():
        out_ref[...] = jnp.zeros((G, D), jnp.float32)

    out_ref[...] += pool


def _pool_call(h1, agg2, wu, bu, batch):
    return pl.pallas_call(
        _pool_body,
        grid=(NGRID,),
        in_specs=[
            pl.BlockSpec((RBLK, D), lambda i: (i, 0)),
            pl.BlockSpec((RBLK, D), lambda i: (i, 0)),
            pl.BlockSpec((D, D), lambda i: (0, 0)),
            pl.BlockSpec((1, D), lambda i: (0, 0)),
            pl.BlockSpec((RBLK, 1), lambda i: (i, 0)),
        ],
        out_specs=pl.BlockSpec((G, D), lambda i: (0, 0)),
        out_shape=jax.ShapeDtypeStruct((G, D), jnp.float32),
    )(h1, agg2, wu, bu, batch)


def _head_body(xl_ref, xr_ref, f_ref, wa_ref, wb_ref, wc_ref, b1_ref,
               w2_ref, b2_ref, out_ref):
    x = (jnp.dot(xl_ref[...], wa_ref[...], preferred_element_type=jnp.float32)
         + jnp.dot(xr_ref[...], wb_ref[...], preferred_element_type=jnp.float32)
         + f_ref[...] * wc_ref[...]
         + b1_ref[...])
    x = _lrelu(x)
    out_ref[...] = jnp.dot(x, w2_ref[...],
                           preferred_element_type=jnp.float32) + b2_ref[...]


def _head_call(xl, xr, force, wa, wb, wc, b1, w2, b2):
    return pl.pallas_call(
        _head_body,
        out_shape=jax.ShapeDtypeStruct((G, 1), jnp.float32),
    )(xl, xr, force, wa, wb, wc, b1, w2, b2)


def kernel(left_species, left_edge_index, left_batch,
           right_species, right_edge_index, right_batch, force,
           emb_L, Wm1_L, bm1_L, Wu1_L, bu1_L, Wm2_L, bm2_L, Wu2_L, bu2_L,
           emb_R, Wm1_R, bm1_R, Wu1_R, bu1_R, Wm2_R, bm2_R, Wu2_R, bu2_R,
           l1_W, l1_b, l2_W, l2_b):
    f32 = jnp.float32
    spL = left_species.astype(jnp.int32).reshape(N, 1)
    spR = right_species.astype(jnp.int32).reshape(N, 1)
    eiL = left_edge_index.astype(jnp.int32)
    eiR = right_edge_index.astype(jnp.int32)
    batL = left_batch.astype(jnp.int32).reshape(N, 1)
    batR = right_batch.astype(jnp.int32).reshape(N, 1)

    embL_p = jnp.zeros((8, D), f32).at[:5].set(emb_L.astype(f32))
    embR_p = jnp.zeros((8, D), f32).at[:5].set(emb_R.astype(f32))
    zrows = jnp.zeros((STRIPE, D), f32)

    def r1(b):
        return b.astype(f32).reshape(1, D)

    hm1L = _hm1_call(spL, embL_p, Wm1_L.astype(f32), r1(bm1_L))
    hm1R = _hm1_call(spR, embR_p, Wm1_R.astype(f32), r1(bm1_R))

    agg1L, agg1R = _edge_pass(eiL, hm1L, eiR, hm1R, zrows)

    h1L, hm2L = _mid_call(spL, embL_p, agg1L, Wu1_L.astype(f32), r1(bu1_L),
                          Wm2_L.astype(f32), r1(bm2_L))
    h1R, hm2R = _mid_call(spR, embR_p, agg1R, Wu1_R.astype(f32), r1(bu1_R),
                          Wm2_R.astype(f32), r1(bm2_R))

    agg2L, agg2R = _edge_pass(eiL, hm2L, eiR, hm2R, zrows)

    pooledL = _pool_call(h1L, agg2L, Wu2_L.astype(f32), r1(bu2_L), batL)
    pooledR = _pool_call(h1R, agg2R, Wu2_R.astype(f32), r1(bu2_R), batR)

    l1W = l1_W.astype(f32)
    out = _head_call(pooledL, pooledR, force.astype(f32),
                     l1W[0:D], l1W[D:2 * D], l1W[2 * D:2 * D + 1],
                     l1_b.astype(f32).reshape(1, D),
                     l2_W.astype(f32), l2_b.astype(f32).reshape(1, 1))
    return out


# speedup vs baseline: 1.4203x; 1.3627x over previous
import functools

import jax
import jax.numpy as jnp
from jax import lax
from jax.experimental import pallas as pl
from jax.experimental.pallas import tpu as pltpu
from jax.experimental.pallas import tpu_sc as plsc

N = 100000
E = 3200000
G = 64
D = 16
NP = 102400
NP8 = NP // 8

NTILES = 16
STRIPE = NP // NTILES
GE = 512
NSG = E // GE
SG_LO = NSG // NTILES
N_HI_TILES2 = NSG - SG_LO * NTILES
NPAIRS = (SG_LO + 1) // 2

RB8 = 1280
NGRID = 2 * NP8 // RB8
PER_SIDE = NGRID // 2
RBLK = RB8 * 8


def _lrelu(x):
    return jnp.where(x >= 0, x, 0.01 * x)


def _edge_body(eiL, eiR, hm_all, zrows, agg_all,
               sbuf0, sbuf1, dbuf0, dbuf1, rows0, rows1, aggsh,
               sem_is0, sem_is1, sem_id0, sem_id1,
               sem_g0, sem_g1, sem_s0, sem_s1):
    c = lax.axis_index("c")
    s = lax.axis_index("s")

    pltpu.sync_copy(zrows, aggsh.at[pl.ds(s * STRIPE, STRIPE)])
    plsc.subcore_barrier()

    sbufs = (sbuf0, sbuf1)
    dbufs = (dbuf0, dbuf1)
    rowss = (rows0, rows1)
    sems_is = (sem_is0, sem_is1)
    sems_id = (sem_id0, sem_id1)
    sems_g = (sem_g0, sem_g1)
    sems_s = (sem_s0, sem_s1)

    def side(ei, row0):
        hm = hm_all.at[pl.ds(row0, NP)]

        def fire_isrc(p, e0):
            pltpu.async_copy(ei.at[0, pl.ds(e0, GE)], sbufs[p], sems_is[p])

        def fire_idst(p, e0):
            pltpu.async_copy(ei.at[1, pl.ds(e0, GE)], dbufs[p], sems_id[p])

        def wait_is(p):
            pltpu.make_async_copy(ei.at[0, pl.ds(0, GE)], sbufs[p],
                                  sems_is[p]).wait()

        def wait_id(p):
            pltpu.make_async_copy(ei.at[1, pl.ds(0, GE)], dbufs[p],
                                  sems_id[p]).wait()

        def fire_g(p):
            pltpu.async_copy(hm.at[sbufs[p]], rowss[p], sems_g[p])

        def wait_g(p):
            pltpu.make_async_copy(hm.at[sbufs[p]], rowss[p],
                                  sems_g[p]).wait()

        def fire_s(p):
            pltpu.async_copy(rowss[p], aggsh.at[dbufs[p]], sems_s[p],
                             add=True)

        def wait_s(p):
            pltpu.make_async_copy(rowss[p], aggsh.at[dbufs[p]],
                                  sems_s[p]).wait()

        base_e = (s * SG_LO + jnp.minimum(s, N_HI_TILES2)) * GE
        has_extra = s < N_HI_TILES2
        ng2 = jnp.where(s < N_HI_TILES2, SG_LO + 1, SG_LO)

        fire_isrc(0, base_e)
        fire_idst(0, base_e)
        fire_isrc(1, base_e + GE)
        fire_idst(1, base_e + GE)
        wait_is(0)
        fire_g(0)

        def body(k, carry):
            a_e = base_e + 2 * k * GE
            next0 = 2 * k + 2 < ng2
            next1 = 2 * k + 3 < ng2
            wait_g(0)

            @pl.when(next0)
            def _():
                fire_isrc(0, a_e + 2 * GE)

            wait_is(1)
            wait_id(1)
            fire_g(1)
            wait_id(0)
            fire_s(0)
            wait_g(1)
            fire_s(1)
            wait_s(0)

            @pl.when(next0)
            def _():
                fire_idst(0, a_e + 2 * GE)
                wait_is(0)
                fire_g(0)

            wait_s(1)

            @pl.when(next1)
            def _():
                fire_isrc(1, a_e + 3 * GE)
                fire_idst(1, a_e + 3 * GE)

            return carry

        lax.fori_loop(0, NPAIRS, body, 0)

        @pl.when(has_extra)
        def _():
            wait_id(0)
            wait_g(0)
            fire_s(0)
            wait_s(0)

    @pl.when(c == 0)
    def _():
        side(eiL, 0)

    @pl.when(c == 1)
    def _():
        side(eiR, NP)

    plsc.subcore_barrier()

    def copy_out(row0):
        r0 = s * STRIPE
        pltpu.sync_copy(aggsh.at[pl.ds(r0, STRIPE)],
                        agg_all.at[pl.ds(row0 + r0, STRIPE)])

    @pl.when(c == 0)
    def _():
        copy_out(0)

    @pl.when(c == 1)
    def _():
        copy_out(NP)


@functools.cache
def _edge_pass_fn():
    return pl.kernel(
        _edge_body,
        out_type=jax.ShapeDtypeStruct((2 * NP, D), jnp.float32),
        mesh=plsc.VectorSubcoreMesh(core_axis_name="c", subcore_axis_name="s",
                                    num_cores=2, num_subcores=NTILES),
        scratch_types=[
            pltpu.VMEM((GE,), jnp.int32),
            pltpu.VMEM((GE,), jnp.int32),
            pltpu.VMEM((GE,), jnp.int32),
            pltpu.VMEM((GE,), jnp.int32),
            pltpu.VMEM((GE, D), jnp.float32),
            pltpu.VMEM((GE, D), jnp.float32),
            pltpu.VMEM_SHARED((NP, D), jnp.float32),
        ] + [pltpu.SemaphoreType.DMA] * 8,
        compiler_params=pltpu.CompilerParams(use_tc_tiling_on_sc=False),
    )


def _edge_pass(*args):
    return _edge_pass_fn()(*args)


def _w_spec():
    return pl.BlockSpec((1, 128, 128), lambda i: (i // PER_SIDE, 0, 0))


def _b_spec():
    return pl.BlockSpec((1, 1, 128), lambda i: (i // PER_SIDE, 0, 0))


def _e_spec():
    return pl.BlockSpec((1, 8, 128), lambda i: (i // PER_SIDE, 0, 0))


def _n_spec():
    return pl.BlockSpec((RB8, 128), lambda i: (i, 0))


def _select_species(sp16_blk, table_t):
    h = jnp.zeros((RB8, 128), jnp.float32)
    for sidx in range(5):
        h = jnp.where(sp16_blk == sidx, table_t[sidx:sidx + 1, :], h)
    return h


def _hm1_body(sp_ref, embt_ref, wbd_ref, bt_ref, out_ref):
    table_t = _lrelu(jnp.dot(embt_ref[0], wbd_ref[0],
                             preferred_element_type=jnp.float32)
                     + bt_ref[0])
    out_ref[...] = _select_species(sp_ref[...], table_t)


def _hm1_call(sp16, embt2, wm_bd2, bm_t2):
    return pl.pallas_call(
        _hm1_body,
        grid=(NGRID,),
        in_specs=[_n_spec(), _e_spec(), _w_spec(), _b_spec()],
        out_specs=_n_spec(),
        out_shape=jax.ShapeDtypeStruct((2 * NP8, 128), jnp.float32),
    )(sp16, embt2, wm_bd2, bm_t2)


def _mid_body(sp_ref, embt_ref, agg_ref, wu_ref, bu_ref, wm_ref, bm_ref,
              h1_ref, hm2_ref):
    h0 = _select_species(sp_ref[...], embt_ref[0])
    upd = _lrelu(jnp.dot(agg_ref[...], wu_ref[0],
                         preferred_element_type=jnp.float32) + bu_ref[0])
    h1 = h0 + upd
    h1_ref[...] = h1
    hm2_ref[...] = _lrelu(jnp.dot(h1, wm_ref[0],
                                  preferred_element_type=jnp.float32)
                          + bm_ref[0])


def _mid_call(sp16, embt2, agg1, wu_bd2, bu_t2, wm_bd2, bm_t2):
    return pl.pallas_call(
        _mid_body,
        grid=(NGRID,),
        in_specs=[_n_spec(), _e_spec(), _n_spec(),
                  _w_spec(), _b_spec(), _w_spec(), _b_spec()],
        out_specs=[_n_spec(), _n_spec()],
        out_shape=[jax.ShapeDtypeStruct((2 * NP8, 128), jnp.float32),
                   jax.ShapeDtypeStruct((2 * NP8, 128), jnp.float32)],
    )(sp16, embt2, agg1, wu_bd2, bu_t2, wm_bd2, bm_t2)


def _pool_body(h1_ref, agg_ref, wu_ref, bu_ref, batch_ref, out_ref):
    i = pl.program_id(0)
    upd = _lrelu(jnp.dot(agg_ref[...], wu_ref[0],
                         preferred_element_type=jnp.float32) + bu_ref[0])
    h2 = h1_ref[...] + upd
    gids = lax.broadcasted_iota(jnp.int32, (1, G), 1)
    mask = (batch_ref[...] == gids).astype(jnp.float32)
    pool = lax.dot_general(mask, h2, (((0,), (0,)), ((), ())),
                           precision=lax.Precision.HIGHEST,
                           preferred_element_type=jnp.float32)

    @pl.when(i % PER_SIDE == 0)
    def _():
        out_ref[...] = jnp.zeros((1, G, D), jnp.float32)

    out_ref[...] += pool[None]


def _pool_call(h1, agg2, wu2, bu2, batch2):
    return pl.pallas_call(
        _pool_body,
        grid=(NGRID,),
        in_specs=[
            pl.BlockSpec((RBLK, D), lambda i: (i, 0)),
            pl.BlockSpec((RBLK, D), lambda i: (i, 0)),
            pl.BlockSpec((1, D, D), lambda i: (i // PER_SIDE, 0, 0)),
            pl.BlockSpec((1, 1, D), lambda i: (i // PER_SIDE, 0, 0)),
            pl.BlockSpec((RBLK, 1), lambda i: (i, 0)),
        ],
        out_specs=pl.BlockSpec((1, G, D), lambda i: (i // PER_SIDE, 0, 0)),
        out_shape=jax.ShapeDtypeStruct((2, G, D), jnp.float32),
    )(h1, agg2, wu2, bu2, batch2)


def _head_body(p_ref, f_ref, wa_ref, wb_ref, wc_ref, b1_ref,
               w2_ref, b2_ref, out_ref):
    x = (jnp.dot(p_ref[0], wa_ref[...], preferred_element_type=jnp.float32)
         + jnp.dot(p_ref[1], wb_ref[...], preferred_element_type=jnp.float32)
         + f_ref[...] * wc_ref[...]
         + b1_ref[...])
    x = _lrelu(x)
    out_ref[...] = jnp.dot(x, w2_ref[...],
                           preferred_element_type=jnp.float32) + b2_ref[...]


def _head_call(pooled, force, wa, wb, wc, b1, w2, b2):
    return pl.pallas_call(
        _head_body,
        out_shape=jax.ShapeDtypeStruct((G, 1), jnp.float32),
    )(pooled, force, wa, wb, wc, b1, w2, b2)


def kernel(left_species, left_edge_index, left_batch,
           right_species, right_edge_index, right_batch, force,
           emb_L, Wm1_L, bm1_L, Wu1_L, bu1_L, Wm2_L, bm2_L, Wu2_L, bu2_L,
           emb_R, Wm1_R, bm1_R, Wu1_R, bu1_R, Wm2_R, bm2_R, Wu2_R, bu2_R,
           l1_W, l1_b, l2_W, l2_b):
    f32 = jnp.float32
    eye8 = jnp.eye(8, dtype=f32)

    def wbd(wl, wr):
        return jnp.stack([jnp.kron(eye8, wl.astype(f32)),
                          jnp.kron(eye8, wr.astype(f32))])

    def bt(bl, br):
        return jnp.stack([jnp.tile(bl.astype(f32).reshape(1, D), (1, 8)),
                          jnp.tile(br.astype(f32).reshape(1, D), (1, 8))])

    def embt(e):
        ep = jnp.zeros((8, D), f32).at[:5].set(e.astype(f32))
        return jnp.tile(ep, (1, 8))

    pad_sp = jnp.full((NP - N,), 5, jnp.int32)
    pad_bat = jnp.full((NP - N,), G, jnp.int32)
    sp_all = jnp.concatenate([left_species.astype(jnp.int32), pad_sp,
                              right_species.astype(jnp.int32), pad_sp])
    sp16 = jnp.repeat(sp_all, D).reshape(2 * NP8, 128)
    batch2 = jnp.concatenate([left_batch.astype(jnp.int32), pad_bat,
                              right_batch.astype(jnp.int32), pad_bat]
                             ).reshape(2 * NP, 1)
    eiL = left_edge_index.astype(jnp.int32)
    eiR = right_edge_index.astype(jnp.int32)
    zrows = jnp.zeros((STRIPE, D), f32)

    embt2 = jnp.stack([embt(emb_L), embt(emb_R)])
    wm1_bd2 = wbd(Wm1_L, Wm1_R)
    bm1_t2 = bt(bm1_L, bm1_R)
    wu1_bd2 = wbd(Wu1_L, Wu1_R)
    bu1_t2 = bt(bu1_L, bu1_R)
    wm2_bd2 = wbd(Wm2_L, Wm2_R)
    bm2_t2 = bt(bm2_L, bm2_R)
    wu2_2 = jnp.stack([Wu2_L.astype(f32), Wu2_R.astype(f32)])
    bu2_2 = jnp.stack([bu2_L.astype(f32).reshape(1, D),
                       bu2_R.astype(f32).reshape(1, D)])

    hm1 = _hm1_call(sp16, embt2, wm1_bd2, bm1_t2)
    agg1 = _edge_pass(eiL, eiR, hm1.reshape(2 * NP, D), zrows)
    h1, hm2 = _mid_call(sp16, embt2, agg1.reshape(2 * NP8, 128),
                        wu1_bd2, bu1_t2, wm2_bd2, bm2_t2)
    agg2 = _edge_pass(eiL, eiR, hm2.reshape(2 * NP, D), zrows)
    pooled = _pool_call(h1.reshape(2 * NP, D), agg2, wu2_2, bu2_2, batch2)

    l1W = l1_W.astype(f32)
    out = _head_call(pooled, force.astype(f32),
                     l1W[0:D], l1W[D:2 * D], l1W[2 * D:2 * D + 1],
                     l1_b.astype(f32).reshape(1, D),
                     l2_W.astype(f32), l2_b.astype(f32).reshape(1, 1))
    return out
